# baseline (speedup 1.0000x reference)
import functools

import jax
import jax.numpy as jnp
from jax.experimental import pallas as pl
from jax.experimental.pallas import tpu as pltpu

_NE = 512
_D = 32
_BETA = 0.25
_B = 4
_THW = 16384
_ROWS = _B * _THW
_R = 1024
_NT = _THW // _R


def _vq_tile(x_ref, e_ref, d_ref, inds_ref, zq_ref, loss_ref):
    b = pl.program_id(0)
    t = pl.program_id(1)
    step = b * _NT + t
    x_cr = x_ref[0]
    e = e_ref[...]
    x = jnp.transpose(x_cr)

    x2 = jnp.sum(x * x, axis=1, keepdims=True)
    e2_full = jax.lax.dot_general(
        jnp.ones((8, _D), jnp.float32), e * e,
        (((1,), (1,)), ((), ())),
        precision=jax.lax.Precision.HIGHEST,
        preferred_element_type=jnp.float32)
    e2 = e2_full[0:1, :]
    ze = jax.lax.dot_general(
        x, e, (((1,), (1,)), ((), ())),
        preferred_element_type=jnp.float32)
    d = (x2 + e2) - 2.0 * ze
    d_ref[...] = d

    dmin = jnp.min(d, axis=1, keepdims=True)
    lane = jax.lax.broadcasted_iota(jnp.int32, (_R, _NE), 1)
    idx = jnp.min(jnp.where(d == dmin, lane, _NE), axis=1,
                  keepdims=True)
    inds_ref[...] = idx

    oh = (lane == idx).astype(jnp.float32)
    zq = jax.lax.dot_general(
        oh, e, (((1,), (0,)), ((), ())),
        precision=jax.lax.Precision.HIGHEST,
        preferred_element_type=jnp.float32)
    zq_st = x + (zq - x)
    zq_ref[0] = jnp.transpose(zq_st)

    diff = zq - x
    part = jnp.sum(diff * diff).reshape(1, 1)

    @pl.when(step == 0)
    def _():
        loss_ref[...] = part

    @pl.when(jnp.logical_and(step > 0, step < _B * _NT - 1))
    def _():
        loss_ref[...] = loss_ref[...] + part

    @pl.when(step == _B * _NT - 1)
    def _():
        total = loss_ref[...] + part
        m = total / jnp.float32(_ROWS * _D)
        loss_ref[...] = m + _BETA * m


@functools.partial(jax.jit, static_argnames=("interpret",))
def kernel(z, E, interpret=False):
    B, C, T, H, W = z.shape
    z3 = z.reshape(B, C, T * H * W)

    d, inds, zq3, loss = pl.pallas_call(
        _vq_tile,
        grid=(_B, _NT),
        in_specs=[
            pl.BlockSpec((1, _D, _R), lambda b, t: (b, 0, t)),
            pl.BlockSpec((_NE, _D), lambda b, t: (0, 0)),
        ],
        out_specs=[
            pl.BlockSpec((_R, _NE), lambda b, t: (b * _NT + t, 0)),
            pl.BlockSpec((_R, 1), lambda b, t: (b * _NT + t, 0)),
            pl.BlockSpec((1, _D, _R), lambda b, t: (b, 0, t)),
            pl.BlockSpec((1, 1), lambda b, t: (0, 0)),
        ],
        out_shape=[
            jax.ShapeDtypeStruct((_ROWS, _NE), jnp.float32),
            jax.ShapeDtypeStruct((_ROWS, 1), jnp.int32),
            jax.ShapeDtypeStruct((_B, _D, _THW), jnp.float32),
            jax.ShapeDtypeStruct((1, 1), jnp.float32),
        ],
        interpret=interpret,
    )(z3, E)

    z_q_st = zq3.reshape(B, C, T, H, W)
    inds_out = inds.reshape(B, T, H, W)
    return z_q_st, loss.reshape(()), inds_out, d


# speedup vs baseline: 1.4657x; 1.4657x over previous
import functools

import jax
import jax.numpy as jnp
from jax.experimental import pallas as pl
from jax.experimental.pallas import tpu as pltpu

_NE = 512
_D = 32
_BETA = 0.25
_B = 4
_THW = 16384
_ROWS = _B * _THW
_R = 1024
_NT = _THW // _R


def _vq_tile(x_ref, e_ref, d_ref, inds_ref, zq_ref, loss_ref):
    b = pl.program_id(0)
    t = pl.program_id(1)
    step = b * _NT + t
    x_cr = x_ref[0]
    e = e_ref[...]
    x = jnp.transpose(x_cr)

    x2 = jnp.sum(x * x, axis=1, keepdims=True)
    e2_full = jax.lax.dot_general(
        jnp.ones((8, _D), jnp.float32), e * e,
        (((1,), (1,)), ((), ())),
        precision=jax.lax.Precision.HIGHEST,
        preferred_element_type=jnp.float32)
    e2 = e2_full[0:1, :]
    ze = jax.lax.dot_general(
        x, e, (((1,), (1,)), ((), ())),
        preferred_element_type=jnp.float32)
    d = (x2 + e2) - 2.0 * ze
    d_ref[...] = d

    dmin = jnp.min(d, axis=1, keepdims=True)
    lane = jax.lax.broadcasted_iota(jnp.int32, (_R, _NE), 1)
    idx = jnp.min(jnp.where(d == dmin, lane, _NE), axis=1,
                  keepdims=True)
    inds_ref[...] = idx

    oh = (lane == idx).astype(jnp.float32)
    zq = jax.lax.dot_general(
        oh, e, (((1,), (0,)), ((), ())),
        preferred_element_type=jnp.float32)
    zq_st = x + (zq - x)
    zq_ref[0] = jnp.transpose(zq_st)

    diff = zq - x
    part = jnp.sum(diff * diff).reshape(1, 1)

    @pl.when(step == 0)
    def _():
        loss_ref[...] = part

    @pl.when(jnp.logical_and(step > 0, step < _B * _NT - 1))
    def _():
        loss_ref[...] = loss_ref[...] + part

    @pl.when(step == _B * _NT - 1)
    def _():
        total = loss_ref[...] + part
        m = total / jnp.float32(_ROWS * _D)
        loss_ref[...] = m + _BETA * m


@functools.partial(jax.jit, static_argnames=("interpret",))
def kernel(z, E, interpret=False):
    B, C, T, H, W = z.shape
    z3 = z.reshape(B, C, T * H * W)

    d, inds, zq3, loss = pl.pallas_call(
        _vq_tile,
        grid=(_B, _NT),
        in_specs=[
            pl.BlockSpec((1, _D, _R), lambda b, t: (b, 0, t)),
            pl.BlockSpec((_NE, _D), lambda b, t: (0, 0)),
        ],
        out_specs=[
            pl.BlockSpec((_R, _NE), lambda b, t: (b * _NT + t, 0)),
            pl.BlockSpec((_R, 1), lambda b, t: (b * _NT + t, 0)),
            pl.BlockSpec((1, _D, _R), lambda b, t: (b, 0, t)),
            pl.BlockSpec((1, 1), lambda b, t: (0, 0)),
        ],
        out_shape=[
            jax.ShapeDtypeStruct((_ROWS, _NE), jnp.float32),
            jax.ShapeDtypeStruct((_ROWS, 1), jnp.int32),
            jax.ShapeDtypeStruct((_B, _D, _THW), jnp.float32),
            jax.ShapeDtypeStruct((1, 1), jnp.float32),
        ],
        interpret=interpret,
    )(z3, E)

    z_q_st = zq3.reshape(B, C, T, H, W)
    inds_out = inds.reshape(B, T, H, W)
    return z_q_st, loss.reshape(()), inds_out, d


# speedup vs baseline: 1.7297x; 1.1802x over previous
import functools

import jax
import jax.numpy as jnp
from jax.experimental import pallas as pl
from jax.experimental.pallas import tpu as pltpu

_NE = 512
_D = 32
_BETA = 0.25
_B = 4
_T = 16
_HW = 1024
_ROWS = _B * _T * _HW
_R = _HW


def _vq_tile(x_ref, e_ref, d_ref, inds_ref, zq_ref, loss_ref):
    b = pl.program_id(0)
    t = pl.program_id(1)
    step = b * _T + t
    x_cr = x_ref[0]
    e = e_ref[...]
    x = jnp.transpose(x_cr)

    x2 = jnp.sum(x * x, axis=1, keepdims=True)
    e2_full = jax.lax.dot_general(
        jnp.ones((8, _D), jnp.float32), e * e,
        (((1,), (1,)), ((), ())),
        precision=jax.lax.Precision.HIGHEST,
        preferred_element_type=jnp.float32)
    e2 = e2_full[0:1, :]
    ze = jax.lax.dot_general(
        x, e, (((1,), (1,)), ((), ())),
        preferred_element_type=jnp.float32)
    d = (x2 + e2) - 2.0 * ze
    d_ref[...] = d

    dmin = jnp.min(d, axis=1, keepdims=True)
    lane = jax.lax.broadcasted_iota(jnp.int32, (_R, _NE), 1)
    idx = jnp.min(jnp.where(d == dmin, lane, _NE), axis=1,
                  keepdims=True)
    inds_ref[...] = idx.reshape(1, 1, 32, 32)

    oh = (lane == idx).astype(jnp.float32)
    zq = jax.lax.dot_general(
        oh, e, (((1,), (0,)), ((), ())),
        preferred_element_type=jnp.float32)
    zq_st = x + (zq - x)
    zq_ref[...] = jnp.transpose(zq_st).reshape(1, _D, 1, 32, 32)

    diff = zq - x
    part = jnp.sum(diff * diff).reshape(1, 1)

    @pl.when(step == 0)
    def _():
        loss_ref[...] = part

    @pl.when(jnp.logical_and(step > 0, step < _B * _T - 1))
    def _():
        loss_ref[...] = loss_ref[...] + part

    @pl.when(step == _B * _T - 1)
    def _():
        total = loss_ref[...] + part
        m = total / jnp.float32(_ROWS * _D)
        loss_ref[...] = m + _BETA * m


@functools.partial(jax.jit, static_argnames=("interpret",))
def kernel(z, E, interpret=False):
    B, C, T, H, W = z.shape
    z3 = z.reshape(B, C, T * H * W)

    d, inds_out, z_q_st, loss = pl.pallas_call(
        _vq_tile,
        grid=(_B, _T),
        in_specs=[
            pl.BlockSpec((1, _D, _R), lambda b, t: (b, 0, t)),
            pl.BlockSpec((_NE, _D), lambda b, t: (0, 0)),
        ],
        out_specs=[
            pl.BlockSpec((_R, _NE), lambda b, t: (b * _T + t, 0)),
            pl.BlockSpec((1, 1, 32, 32), lambda b, t: (b, t, 0, 0)),
            pl.BlockSpec((1, _D, 1, 32, 32), lambda b, t: (b, 0, t, 0, 0)),
            pl.BlockSpec((1, 1), lambda b, t: (0, 0)),
        ],
        out_shape=[
            jax.ShapeDtypeStruct((_ROWS, _NE), jnp.float32),
            jax.ShapeDtypeStruct((_B, _T, 32, 32), jnp.int32),
            jax.ShapeDtypeStruct((_B, _D, _T, 32, 32), jnp.float32),
            jax.ShapeDtypeStruct((1, 1), jnp.float32),
        ],
        interpret=interpret,
    )(z3, E)

    return z_q_st, loss.reshape(()), inds_out, d


# speedup vs baseline: 1.9608x; 1.1336x over previous
import functools

import jax
import jax.numpy as jnp
from jax.experimental import pallas as pl
from jax.experimental.pallas import tpu as pltpu

_NE = 512
_D = 32
_BETA = 0.25
_B = 4
_T = 16
_HW = 1024
_ROWS = _B * _T * _HW
_R = _HW


def _vq_tile(x_ref, e_ref, d_ref, inds_ref, zq_ref, loss_ref):
    b = pl.program_id(0)
    t = pl.program_id(1)
    step = b * _T + t
    x_cr = x_ref[0, :, 0].reshape(_D, _R)
    e = e_ref[...]
    x = jnp.transpose(x_cr)

    x2 = jnp.sum(x * x, axis=1, keepdims=True)
    e2_full = jax.lax.dot_general(
        jnp.ones((8, _D), jnp.float32), e * e,
        (((1,), (1,)), ((), ())),
        precision=jax.lax.Precision.HIGHEST,
        preferred_element_type=jnp.float32)
    e2 = e2_full[0:1, :]
    ze = jax.lax.dot_general(
        x, e, (((1,), (1,)), ((), ())),
        preferred_element_type=jnp.float32)
    d = (x2 + e2) - 2.0 * ze
    d_ref[...] = d

    dmin = jnp.min(d, axis=1, keepdims=True)
    lane = jax.lax.broadcasted_iota(jnp.int32, (_R, _NE), 1)
    idx = jnp.min(jnp.where(d == dmin, lane, _NE), axis=1,
                  keepdims=True)
    inds_ref[...] = idx.reshape(1, 1, 32, 32)

    oh = (lane == idx).astype(jnp.float32)
    zq = jax.lax.dot_general(
        oh, e, (((1,), (0,)), ((), ())),
        preferred_element_type=jnp.float32)
    zq_st = x + (zq - x)
    zq_ref[...] = jnp.transpose(zq_st).reshape(1, _D, 1, 32, 32)

    diff = zq - x
    part = jnp.sum(diff * diff).reshape(1, 1)

    @pl.when(step == 0)
    def _():
        loss_ref[...] = part

    @pl.when(jnp.logical_and(step > 0, step < _B * _T - 1))
    def _():
        loss_ref[...] = loss_ref[...] + part

    @pl.when(step == _B * _T - 1)
    def _():
        total = loss_ref[...] + part
        m = total / jnp.float32(_ROWS * _D)
        loss_ref[...] = m + _BETA * m


@functools.partial(jax.jit, static_argnames=("interpret",))
def kernel(z, E, interpret=False):
    B, C, T, H, W = z.shape

    d, inds_out, z_q_st, loss = pl.pallas_call(
        _vq_tile,
        grid=(_B, _T),
        in_specs=[
            pl.BlockSpec((1, _D, 1, 32, 32), lambda b, t: (b, 0, t, 0, 0)),
            pl.BlockSpec((_NE, _D), lambda b, t: (0, 0)),
        ],
        out_specs=[
            pl.BlockSpec((_R, _NE), lambda b, t: (b * _T + t, 0)),
            pl.BlockSpec((1, 1, 32, 32), lambda b, t: (b, t, 0, 0)),
            pl.BlockSpec((1, _D, 1, 32, 32), lambda b, t: (b, 0, t, 0, 0)),
            pl.BlockSpec((1, 1), lambda b, t: (0, 0)),
        ],
        out_shape=[
            jax.ShapeDtypeStruct((_ROWS, _NE), jnp.float32),
            jax.ShapeDtypeStruct((_B, _T, 32, 32), jnp.int32),
            jax.ShapeDtypeStruct((_B, _D, _T, 32, 32), jnp.float32),
            jax.ShapeDtypeStruct((1, 1), jnp.float32),
        ],
        interpret=interpret,
    )(z, E)

    return z_q_st, loss.reshape(()), inds_out, d
